# baseline (speedup 1.0000x reference)
import functools

import jax
import jax.numpy as jnp
from jax import lax
from jax.experimental import pallas as pl
from jax.experimental.pallas import tpu as pltpu
from jax.experimental.pallas import tpu_sc as plsc

VOCAB = 1000000
RBLK = 16
NRB = (VOCAB + 1024 * RBLK - 1) // (1024 * RBLK)
VOCAB_PAD = NRB * 1024 * RBLK
D = 64
S = 50
NB = 16384 // 128
NBLK = S * NB
NW = 32
BLK_PER_W = NBLK // NW
SCALE = float(D) ** 0.5

_MESH = plsc.VectorSubcoreMesh(core_axis_name="c", subcore_axis_name="s")


def _repack_kernel(t_ref, o_ref):
    for u in range(RBLK):
        y = t_ref[:, 1024 * u:1024 * (u + 1)].T
        o_ref[512 * u:512 * (u + 1), :] = jnp.concatenate(
            [y[:512], y[512:]], axis=1) * SCALE


def _repack_table(tt):
    return pl.pallas_call(
        _repack_kernel,
        grid=(NRB,),
        in_specs=[pl.BlockSpec((64, 1024 * RBLK), lambda c: (0, c))],
        out_specs=pl.BlockSpec((512 * RBLK, 128), lambda c: (c, 0)),
        out_shape=jax.ShapeDtypeStruct((VOCAB_PAD // 2, 128), jnp.float32),
    )(tt)


def _remap(r):
    off = r % 1024
    return (r - off) + 2 * (off % 512) + off // 512


NBATCH = 16384
PADS = 56
B_PER_W = NBATCH // NW


@functools.partial(
    pl.kernel,
    out_type=jax.ShapeDtypeStruct((NBATCH * PADS, 128), jnp.float32),
    mesh=_MESH,
    compiler_params=pltpu.CompilerParams(
        use_tc_tiling_on_sc=False, needs_layout_passes=False),
    scratch_types=[
        pltpu.VMEM((B_PER_W, PADS), jnp.int32),
        pltpu.VMEM((PADS, D), jnp.float32),
        pltpu.VMEM((PADS, D), jnp.float32),
        pltpu.SemaphoreType.DMA,
        pltpu.SemaphoreType.DMA,
    ],
)
def _gather(xb_hbm, table_hbm, out_hbm, idx_v, g0, g1, semg0, semg1):
    wid = lax.axis_index("s") * 2 + lax.axis_index("c")
    base_b = wid * B_PER_W

    pltpu.sync_copy(xb_hbm.at[pl.ds(base_b, B_PER_W)], idx_v)

    def idx_row(n):
        return idx_v.at[n]

    def store(gbuf, n):
        pltpu.sync_copy(
            gbuf, out_hbm.at[pl.ds((base_b + n) * PADS, PADS), pl.ds(0, D)])

    pltpu.async_copy(table_hbm.at[idx_row(0)], g0, semg0)

    def pair(g, carry):
        n0 = 2 * g
        pltpu.async_copy(table_hbm.at[idx_row(n0 + 1)], g1, semg1)
        pltpu.make_async_copy(table_hbm.at[idx_row(0)], g0, semg0).wait()
        store(g0, n0)
        nxt = jnp.minimum(n0 + 2, B_PER_W - 1)
        pltpu.async_copy(table_hbm.at[idx_row(nxt)], g0, semg0)
        pltpu.make_async_copy(table_hbm.at[idx_row(0)], g1, semg1).wait()
        store(g1, n0 + 1)
        return carry

    lax.fori_loop(0, B_PER_W // 2, pair, 0)

    pltpu.make_async_copy(table_hbm.at[idx_row(0)], g0, semg0).wait()


def kernel(x, table):
    xb = _remap(jnp.pad(x.astype(jnp.int32), ((0, 0), (0, PADS - S))))

    tt = table.T
    t2 = _repack_table(tt)
    tlin = t2.reshape(VOCAB_PAD, D)
    g = _gather(xb, tlin)
    return g.reshape(NBATCH, PADS, 128)[:, :S, :D]


# speedup vs baseline: 3.0032x; 3.0032x over previous
import functools

import jax
import jax.numpy as jnp
from jax import lax
from jax.experimental import pallas as pl
from jax.experimental.pallas import tpu as pltpu
from jax.experimental.pallas import tpu_sc as plsc

VOCAB = 1000000
RBLK = 16
NRB = (VOCAB + 1024 * RBLK - 1) // (1024 * RBLK)
VOCAB_PAD = NRB * 1024 * RBLK
D = 64
S = 50
NB = 16384 // 128
NBLK = S * NB
NW = 32
BLK_PER_W = NBLK // NW
SCALE = float(D) ** 0.5

_MESH = plsc.VectorSubcoreMesh(core_axis_name="c", subcore_axis_name="s")


def _repack_kernel(t_ref, o_ref):
    for u in range(RBLK):
        y = t_ref[:, 1024 * u:1024 * (u + 1)].T
        o_ref[512 * u:512 * (u + 1), :] = jnp.concatenate(
            [y[:512], y[512:]], axis=1) * SCALE


def _repack_table(tt):
    return pl.pallas_call(
        _repack_kernel,
        grid=(NRB,),
        in_specs=[pl.BlockSpec((64, 1024 * RBLK), lambda c: (0, c))],
        out_specs=pl.BlockSpec((512 * RBLK, 128), lambda c: (c, 0)),
        out_shape=jax.ShapeDtypeStruct((VOCAB_PAD // 2, 128), jnp.float32),
    )(tt)


def _remap(r):
    off = r % 1024
    return (r - off) + 2 * (off % 512) + off // 512


@functools.partial(
    pl.kernel,
    out_type=jax.ShapeDtypeStruct((NBLK * 128, D), jnp.float32),
    mesh=_MESH,
    compiler_params=pltpu.CompilerParams(
        use_tc_tiling_on_sc=False, needs_layout_passes=False),
    scratch_types=[
        pltpu.VMEM((BLK_PER_W, 128), jnp.int32),
        pltpu.VMEM((128, D), jnp.float32),
        pltpu.VMEM((128, D), jnp.float32),
        pltpu.SemaphoreType.DMA,
        pltpu.SemaphoreType.DMA,
    ],
)
def _gather(xb_hbm, table_hbm, out_hbm, idx_v, g0, g1, semg0, semg1):
    wid = lax.axis_index("s") * 2 + lax.axis_index("c")
    base_blk = wid * BLK_PER_W

    pltpu.sync_copy(xb_hbm.at[pl.ds(base_blk, BLK_PER_W)], idx_v)

    def store(gbuf, n):
        pltpu.sync_copy(gbuf, out_hbm.at[pl.ds((base_blk + n) * 128, 128)])

    pltpu.async_copy(table_hbm.at[idx_v.at[0]], g0, semg0)

    def pair(g, carry):
        n0 = 2 * g
        pltpu.async_copy(table_hbm.at[idx_v.at[n0 + 1]], g1, semg1)
        pltpu.make_async_copy(table_hbm.at[idx_v.at[0]], g0, semg0).wait()
        store(g0, n0)
        nxt = jnp.minimum(n0 + 2, BLK_PER_W - 1)
        pltpu.async_copy(table_hbm.at[idx_v.at[nxt]], g0, semg0)
        pltpu.make_async_copy(table_hbm.at[idx_v.at[0]], g1, semg1).wait()
        store(g1, n0 + 1)
        return carry

    lax.fori_loop(0, BLK_PER_W // 2, pair, 0)

    pltpu.make_async_copy(table_hbm.at[idx_v.at[0]], g0, semg0).wait()


def kernel(x, table):
    xb = _remap(x.reshape(NBLK, 128).astype(jnp.int32))

    tt = table.T
    t2 = _repack_table(tt)
    tlin = t2.reshape(VOCAB_PAD, D)
    g = _gather(xb, tlin)
    return g.reshape(16384, S, D)


# speedup vs baseline: 3.0510x; 1.0159x over previous
import functools

import jax
import jax.numpy as jnp
from jax import lax
from jax.experimental import pallas as pl
from jax.experimental.pallas import tpu as pltpu
from jax.experimental.pallas import tpu_sc as plsc

VOCAB = 1000000
RBLK = 32
NRB = (VOCAB + 1024 * RBLK - 1) // (1024 * RBLK)
VOCAB_PAD = NRB * 1024 * RBLK
D = 64
S = 50
NB = 16384 // 128
NBLK = S * NB
NW = 32
BLK_PER_W = NBLK // NW
SCALE = float(D) ** 0.5

_MESH = plsc.VectorSubcoreMesh(core_axis_name="c", subcore_axis_name="s")


def _repack_kernel(t_ref, o_ref):
    for u in range(RBLK):
        y = t_ref[:, 1024 * u:1024 * (u + 1)].T
        o_ref[512 * u:512 * (u + 1), :] = jnp.concatenate(
            [y[:512], y[512:]], axis=1) * SCALE


def _repack_table(tt):
    return pl.pallas_call(
        _repack_kernel,
        grid=(NRB,),
        in_specs=[pl.BlockSpec((64, 1024 * RBLK), lambda c: (0, c))],
        out_specs=pl.BlockSpec((512 * RBLK, 128), lambda c: (c, 0)),
        out_shape=jax.ShapeDtypeStruct((VOCAB_PAD // 2, 128), jnp.float32),
    )(tt)


def _remap(r):
    off = r % 1024
    return (r - off) + 2 * (off % 512) + off // 512


@functools.partial(
    pl.kernel,
    out_type=jax.ShapeDtypeStruct((NBLK * 128, D), jnp.float32),
    mesh=_MESH,
    compiler_params=pltpu.CompilerParams(
        use_tc_tiling_on_sc=False, needs_layout_passes=False),
    scratch_types=[
        pltpu.VMEM((BLK_PER_W, 128), jnp.int32),
        pltpu.VMEM((128, D), jnp.float32),
        pltpu.VMEM((128, D), jnp.float32),
        pltpu.SemaphoreType.DMA,
        pltpu.SemaphoreType.DMA,
    ],
)
def _gather(xb_hbm, table_hbm, out_hbm, idx_v, g0, g1, semg0, semg1):
    wid = lax.axis_index("s") * 2 + lax.axis_index("c")
    base_blk = wid * BLK_PER_W

    pltpu.sync_copy(xb_hbm.at[pl.ds(base_blk, BLK_PER_W)], idx_v)

    def store(gbuf, n):
        pltpu.sync_copy(gbuf, out_hbm.at[pl.ds((base_blk + n) * 128, 128)])

    pltpu.async_copy(table_hbm.at[idx_v.at[0]], g0, semg0)

    def pair(g, carry):
        n0 = 2 * g
        pltpu.async_copy(table_hbm.at[idx_v.at[n0 + 1]], g1, semg1)
        pltpu.make_async_copy(table_hbm.at[idx_v.at[0]], g0, semg0).wait()
        store(g0, n0)
        nxt = jnp.minimum(n0 + 2, BLK_PER_W - 1)
        pltpu.async_copy(table_hbm.at[idx_v.at[nxt]], g0, semg0)
        pltpu.make_async_copy(table_hbm.at[idx_v.at[0]], g1, semg1).wait()
        store(g1, n0 + 1)
        return carry

    lax.fori_loop(0, BLK_PER_W // 2, pair, 0)

    pltpu.make_async_copy(table_hbm.at[idx_v.at[0]], g0, semg0).wait()


def kernel(x, table):
    xb = _remap(x.reshape(NBLK, 128).astype(jnp.int32))

    tt = table.T
    t2 = _repack_table(tt)
    tlin = t2.reshape(VOCAB_PAD, D)
    g = _gather(xb, tlin)
    return g.reshape(16384, S, D)
